# baseline (speedup 1.0000x reference)
import functools

import jax
import jax.numpy as jnp
from jax import lax
from jax.experimental import pallas as pl
from jax.experimental.pallas import tpu as pltpu
from jax.experimental.pallas import tpu_sc as plsc

U_SIZE = 1000000
I_SIZE = 100000
DIM = 64
BATCH = 16384

NUM_CORES = 2
NUM_SUBCORES = 16
NUM_WORKERS = NUM_CORES * NUM_SUBCORES
ROWS_PER_WORKER = BATCH // NUM_WORKERS
STAGE_ROWS = BATCH + 128
CAP = BATCH
NIDX_VECS = BATCH // 16
LANES = 16

_COMPILER_PARAMS = pltpu.CompilerParams(
    needs_layout_passes=False, use_tc_tiling_on_sc=True)


def _make_extract(n_rows):
  nb = n_rows // 128
  ts = nb * 128
  tailn = n_rows - ts
  mesh = plsc.VectorSubcoreMesh(core_axis_name="c", subcore_axis_name="s")

  @functools.partial(
      pl.kernel,
      mesh=mesh,
      out_type=jax.ShapeDtypeStruct((STAGE_ROWS, 2 * DIM), jnp.float32),
      compiler_params=_COMPILER_PARAMS,
      scratch_types=[
          pltpu.VMEM((BATCH,), jnp.int32),
          pltpu.VMEM((CAP + 16,), jnp.int32),
          pltpu.VMEM((CAP + 16,), jnp.int32),
          pltpu.VMEM((64, 128), jnp.float32),
          pltpu.VMEM((tailn, 2 * DIM), jnp.float32),
          pltpu.VMEM((128, 2 * DIM), jnp.float32),
          pltpu.VMEM((CAP // 128, 128), jnp.int32),
          pltpu.SemaphoreType.DMA,
      ],
  )
  def k(idx_hbm, ut_hbm, tail_hbm, rows_hbm,
        idx_v, myu_v, mypos_v, vbuf, tbuf, lrows, lpos_v, sem):
    wid = lax.axis_index("s") * NUM_CORES + lax.axis_index("c")
    blk0 = (wid * nb) >> 5
    blk1 = ((wid + 1) * nb) >> 5
    is_last = wid == NUM_WORKERS - 1
    lanes = lax.iota(jnp.int32, LANES)
    safe_pos = jnp.full((LANES,), BATCH, jnp.int32)

    def init_body(j, _):
      def init_inner(t, _t):
        plsc.store_scatter(
            lpos_v, [jnp.full((LANES,), j, jnp.int32), t * 16 + lanes],
            safe_pos)
        return _t
      lax.fori_loop(0, 128 // 16, init_inner, 0, unroll=True)
      return _
    lax.fori_loop(0, CAP // 128, init_body, 0, unroll=False)

    pltpu.sync_copy(idx_hbm, idx_v)

    def fbody(i, ptr):
      uvec = idx_v[pl.ds(i * 16, 16)]
      q = lax.shift_right_logical(uvec, 7)
      m = (q >= blk0) & (q < blk1)
      m = m | (is_last & (uvec >= ts))
      plsc.store_compressed(myu_v.at[pl.ds(ptr, 16)], uvec, mask=m)
      plsc.store_compressed(mypos_v.at[pl.ds(ptr, 16)], i * 16 + lanes,
                            mask=m)
      return ptr + jnp.sum(m.astype(jnp.int32))
    nmine = lax.fori_loop(0, NIDX_VECS, fbody, 0, unroll=False)
    nvec = (nmine + 15) >> 4

    fvecs = [c * 16 + lanes for c in range(DIM // 16)]

    def extract_matches(vec_i, b, m, from_tail):
      uvec = myu_v[pl.ds(vec_i * 16, 16)]
      pvec = mypos_v[pl.ds(vec_i * 16, 16)]
      valid = (vec_i * 16 + lanes) < nmine
      if from_tail:
        match = valid & (uvec >= ts)
      else:
        match = valid & (lax.shift_right_logical(uvec, 7) == b)

      def wcond(carry):
        mm, _ = carry
        return jnp.any(mm)

      def wbody(carry):
        mm, m_ = carry
        ff = plsc.all_reduce_ffs(mm)
        sel = lanes == ff
        u_e = jnp.sum(jnp.where(sel, uvec, 0))
        p_e = jnp.sum(jnp.where(sel, pvec, 0))
        slot = m_ & 127
        chunk = m_ >> 7
        if from_tail:
          uloc_v = jnp.full((LANES,), u_e - ts, jnp.int32)
          for c in range(DIM // 16):
            val = plsc.load_gather(tbuf, [uloc_v, fvecs[c]])
            lrows[slot, pl.ds(c * 16, 16)] = val
        else:
          uloc_v = jnp.full((LANES,), u_e & 127, jnp.int32)
          for c in range(DIM // 16):
            val = plsc.load_gather(vbuf, [fvecs[c], uloc_v])
            lrows[slot, pl.ds(c * 16, 16)] = val
        plsc.store_scatter(
            lpos_v,
            [jnp.full((LANES,), chunk, jnp.int32),
             jnp.full((LANES,), slot, jnp.int32)],
            jnp.full((LANES,), p_e, jnp.int32),
            mask=lanes == 0)

        @pl.when(slot == 127)
        def _flush():
          pltpu.async_copy(lrows, rows_hbm.at[lpos_v.at[chunk]], sem).wait()

        return mm & (lanes != ff), m_ + 1

      _, m_new = lax.while_loop(wcond, wbody, (match, m))
      return m_new

    def block_body(b, m):
      pltpu.sync_copy(ut_hbm.at[:, pl.ds(b * 128, 128)], vbuf)

      def vloop(v, m_):
        return extract_matches(v, b, m_, from_tail=False)
      return lax.fori_loop(0, nvec, vloop, m, unroll=False)

    m_total = lax.fori_loop(blk0, blk1, block_body, 0, unroll=False)

    @pl.when(is_last)
    def _tail():
      pltpu.sync_copy(tail_hbm, tbuf)

    def tail_loop(v, m_):
      return extract_matches(v, 0, m_, from_tail=True)
    m_total2 = lax.cond(
        is_last,
        lambda m_: lax.fori_loop(0, nvec, tail_loop, m_, unroll=False),
        lambda m_: m_,
        m_total)

    @pl.when((m_total2 & 127) != 0)
    def _final_flush():
      chunk = m_total2 >> 7
      pltpu.async_copy(lrows, rows_hbm.at[lpos_v.at[chunk]], sem).wait()

  return k


def _make_dot():
  mesh = plsc.VectorSubcoreMesh(core_axis_name="c", subcore_axis_name="s")
  chunk = 128
  n_chunks = ROWS_PER_WORKER // chunk

  @functools.partial(
      pl.kernel,
      mesh=mesh,
      out_type=jax.ShapeDtypeStruct((BATCH,), jnp.float32),
      compiler_params=_COMPILER_PARAMS,
      scratch_types=[
          pltpu.VMEM((chunk, 2 * DIM), jnp.float32),
          pltpu.VMEM((chunk, 2 * DIM), jnp.float32),
          pltpu.VMEM((ROWS_PER_WORKER,), jnp.float32),
          pltpu.SemaphoreType.DMA,
      ],
  )
  def k(rows_u_hbm, rows_i_hbm, out_hbm, ubuf, ibuf, out_v, sem):
    wid = lax.axis_index("s") * NUM_CORES + lax.axis_index("c")
    base = wid * ROWS_PER_WORKER
    lanes = lax.iota(jnp.int32, LANES)

    def chunk_body(c, _):
      row0 = base + c * chunk
      cu = pltpu.async_copy(rows_u_hbm.at[pl.ds(row0, chunk)], ubuf, sem)
      ci = pltpu.async_copy(rows_i_hbm.at[pl.ds(row0, chunk)], ibuf, sem)
      cu.wait()
      ci.wait()

      def group_body(g, _g):
        j_vec = g * 16 + lanes
        acc = jnp.zeros((16,), jnp.float32)
        for d in range(DIM):
          col = (lanes + d) & (DIM - 1)
          ug = plsc.load_gather(ubuf, [j_vec, col])
          ig = plsc.load_gather(ibuf, [j_vec, col])
          acc = acc + ug * ig
        out_v[pl.ds(c * chunk + g * 16, 16)] = acc
        return _g
      lax.fori_loop(0, chunk // 16, group_body, 0, unroll=False)
      return _

    lax.fori_loop(0, n_chunks, chunk_body, 0, unroll=False)
    pltpu.sync_copy(out_v, out_hbm.at[pl.ds(base, ROWS_PER_WORKER)])

  return k


_extract_u = _make_extract(U_SIZE)
_extract_i = _make_extract(I_SIZE)
_dot = _make_dot()

_U_TS = (U_SIZE // 128) * 128
_I_TS = (I_SIZE // 128) * 128


@jax.jit
def kernel(users, items, user_emb, item_emb):
  tail_u = jnp.pad(user_emb[_U_TS:], ((0, 0), (0, DIM)))
  tail_i = jnp.pad(item_emb[_I_TS:], ((0, 0), (0, DIM)))
  rows_u = _extract_u(users, user_emb.T, tail_u)
  rows_i = _extract_i(items, item_emb.T, tail_i)
  return _dot(rows_u, rows_i)


# speedup vs baseline: 1.5271x; 1.5271x over previous
import functools

import jax
import jax.numpy as jnp
from jax import lax
from jax.experimental import pallas as pl
from jax.experimental.pallas import tpu as pltpu
from jax.experimental.pallas import tpu_sc as plsc

U_SIZE = 1000000
I_SIZE = 100000
DIM = 64
BATCH = 16384

NUM_CORES = 2
NUM_SUBCORES = 16
NUM_WORKERS = NUM_CORES * NUM_SUBCORES
ROWS_PER_WORKER = BATCH // NUM_WORKERS
STAGE_ROWS = BATCH + 128
CAP = BATCH
NIDX_VECS = BATCH // 16
LANES = 16
NREG = 16

_COMPILER_PARAMS = pltpu.CompilerParams(
    needs_layout_passes=False, use_tc_tiling_on_sc=True)


def _make_extract(n_rows):
  nb = n_rows // 128
  ts = nb * 128
  tailn = n_rows - ts
  max_wblocks = -(-nb // NUM_WORKERS) + 1
  shift = max(0, (-(-max_wblocks // NREG) - 1).bit_length())
  mesh = plsc.VectorSubcoreMesh(core_axis_name="c", subcore_axis_name="s")

  @functools.partial(
      pl.kernel,
      mesh=mesh,
      out_type=jax.ShapeDtypeStruct((STAGE_ROWS, 2 * DIM), jnp.float32),
      compiler_params=_COMPILER_PARAMS,
      scratch_types=[
          pltpu.VMEM((BATCH,), jnp.int32),
          pltpu.VMEM((CAP + 16,), jnp.int32),
          pltpu.VMEM((CAP + 16,), jnp.int32),
          pltpu.VMEM((CAP,), jnp.int32),
          pltpu.VMEM((64, 128), jnp.float32),
          pltpu.VMEM((64, 128), jnp.float32),
          pltpu.VMEM((tailn, 2 * DIM), jnp.float32),
          pltpu.VMEM((128, 2 * DIM), jnp.float32),
          pltpu.VMEM((CAP // 128, 128), jnp.int32),
          pltpu.SemaphoreType.DMA,
          pltpu.SemaphoreType.DMA,
          pltpu.SemaphoreType.DMA,
      ],
  )
  def k(idx_hbm, ut_hbm, tail_hbm, rows_hbm,
        idx_v, myu_v, mypos_v, bpos_v, vbuf0, vbuf1, tbuf, lrows, lpos_v,
        sem0, sem1, semw):
    wid = lax.axis_index("s") * NUM_CORES + lax.axis_index("c")
    blk0 = (wid * nb) >> 5
    blk1 = ((wid + 1) * nb) >> 5
    is_last = wid == NUM_WORKERS - 1
    lanes = lax.iota(jnp.int32, LANES)
    safe_pos = jnp.full((LANES,), BATCH, jnp.int32)

    def init_body(j, _):
      for t in range(128 // 16):
        lpos_v[j, pl.ds(t * 16, 16)] = safe_pos
      return _
    lax.fori_loop(0, CAP // 128, init_body, 0, unroll=False)

    pltpu.sync_copy(idx_hbm, idx_v)

    def fbody(i, ptr):
      uvec = idx_v[pl.ds(i * 16, 16)]
      q = lax.shift_right_logical(uvec, 7)
      m = (q >= blk0) & (q < blk1)
      m = m | (is_last & (uvec >= ts))
      plsc.store_compressed(myu_v.at[pl.ds(ptr, 16)], uvec, mask=m)
      plsc.store_compressed(mypos_v.at[pl.ds(ptr, 16)], i * 16 + lanes,
                            mask=m)
      return ptr + jnp.sum(m.astype(jnp.int32))
    nmine = lax.fori_loop(0, NIDX_VECS, fbody, 0, unroll=False)
    nvec = (nmine + 15) >> 4

    def region_of(uvec):
      r = lax.shift_right_logical(
          lax.shift_right_logical(uvec, 7) - blk0, shift)
      return jnp.minimum(r, NREG - 1)

    def cbody(v, cnts):
      uvec = myu_v[pl.ds(v * 16, 16)]
      valid = (v * 16 + lanes) < nmine
      r = region_of(uvec)
      for reg in range(NREG):
        pc = plsc.all_reduce_population_count((r == reg) & valid)
        cnts = cnts + jnp.where(lanes == reg, pc, 0)
      return cnts
    cnts_v = lax.fori_loop(0, nvec, cbody, jnp.zeros((LANES,), jnp.int32),
                           unroll=False)
    starts0_v = plsc.cumsum(cnts_v) - cnts_v

    def bbody(v, starts):
      uvec = myu_v[pl.ds(v * 16, 16)]
      pvec = mypos_v[pl.ds(v * 16, 16)]
      valid = (v * 16 + lanes) < nmine
      r = region_of(uvec)
      for reg in range(NREG):
        m = (r == reg) & valid
        ptr = jnp.sum(jnp.where(lanes == reg, starts, 0))
        plsc.store_compressed(idx_v.at[pl.ds(ptr, 16)], uvec, mask=m)
        plsc.store_compressed(bpos_v.at[pl.ds(ptr, 16)], pvec, mask=m)
        pc = plsc.all_reduce_population_count(m)
        starts = starts + jnp.where(lanes == reg, pc, 0)
      return starts
    lax.fori_loop(0, nvec, bbody, starts0_v, unroll=False)

    fvecs = [c * 16 + lanes for c in range(DIM // 16)]

    def extract_matches(vec_i, b, m, rs, re, vbuf, from_tail):
      uvec = idx_v[pl.ds(vec_i * 16, 16)]
      pvec = bpos_v[pl.ds(vec_i * 16, 16)]
      gidx = vec_i * 16 + lanes
      if from_tail:
        match = ((gidx < nmine) & (uvec >= ts))
      else:
        valid = (gidx >= rs) & (gidx < re)
        match = valid & (lax.shift_right_logical(uvec, 7) == b)

      def wcond(carry):
        mm, _ = carry
        return jnp.any(mm)

      def wbody(carry):
        mm, m_ = carry
        ff = plsc.all_reduce_ffs(mm)
        sel = lanes == ff
        u_e = jnp.sum(jnp.where(sel, uvec, 0))
        slot = m_ & 127
        chunk = m_ >> 7
        if from_tail:
          uloc_v = jnp.full((LANES,), u_e - ts, jnp.int32)
          for c in range(DIM // 16):
            val = plsc.load_gather(tbuf, [uloc_v, fvecs[c]])
            lrows[slot, pl.ds(c * 16, 16)] = val
        else:
          uloc_v = jnp.full((LANES,), u_e & 127, jnp.int32)
          for c in range(DIM // 16):
            val = plsc.load_gather(vbuf, [fvecs[c], uloc_v])
            lrows[slot, pl.ds(c * 16, 16)] = val
        plsc.store_scatter(
            lpos_v,
            [jnp.full((LANES,), chunk, jnp.int32),
             jnp.full((LANES,), slot, jnp.int32)],
            pvec, mask=sel)

        @pl.when(slot == 127)
        def _flush():
          pltpu.async_copy(lrows, rows_hbm.at[lpos_v.at[chunk]],
                           semw).wait()

        return mm & (lanes != ff), m_ + 1

      _, m_new = lax.while_loop(wcond, wbody, (match, m))
      return m_new

    def scan_block(b, vbuf, m):
      reg = jnp.minimum(
          lax.shift_right_logical(b - blk0, shift), NREG - 1)
      rs = jnp.sum(jnp.where(lanes == reg, starts0_v, 0))
      re = rs + jnp.sum(jnp.where(lanes == reg, cnts_v, 0))

      def vloop(v, m_):
        return extract_matches(v, b, m_, rs, re, vbuf, from_tail=False)
      return lax.fori_loop(rs >> 4, (re + 15) >> 4, vloop, m, unroll=False)

    def start_copy(b, vbuf, sem):
      return pltpu.async_copy(ut_hbm.at[:, pl.ds(b * 128, 128)], vbuf, sem)

    def wait_copy(vbuf, sem):
      pltpu.make_async_copy(ut_hbm.at[:, pl.ds(0, 128)], vbuf, sem).wait()

    @pl.when(blk1 > blk0)
    def _prime():
      start_copy(blk0, vbuf0, sem0)

    def pair_body(p, m):
      b0 = blk0 + 2 * p
      b1 = b0 + 1
      wait_copy(vbuf0, sem0)

      @pl.when(b1 < blk1)
      def _start_odd():
        start_copy(b1, vbuf1, sem1)

      m = scan_block(b0, vbuf0, m)

      def odd_branch(m_):
        wait_copy(vbuf1, sem1)

        @pl.when(b0 + 2 < blk1)
        def _start_next_even():
          start_copy(b0 + 2, vbuf0, sem0)

        return scan_block(b1, vbuf1, m_)

      return lax.cond(b1 < blk1, odd_branch, lambda m_: m_, m)

    m_total = lax.fori_loop(0, (blk1 - blk0 + 1) >> 1, pair_body, 0,
                            unroll=False)

    @pl.when(is_last)
    def _tail_copy():
      pltpu.sync_copy(tail_hbm, tbuf)

    def tail_loop(v, m_):
      return extract_matches(v, 0, m_, 0, 0, vbuf0, from_tail=True)
    m_total = lax.cond(
        is_last,
        lambda m_: lax.fori_loop(0, nvec, tail_loop, m_, unroll=False),
        lambda m_: m_,
        m_total)

    @pl.when((m_total & 127) != 0)
    def _final_flush():
      pltpu.async_copy(lrows, rows_hbm.at[lpos_v.at[m_total >> 7]],
                       semw).wait()

  return k


def _make_dot():
  mesh = plsc.VectorSubcoreMesh(core_axis_name="c", subcore_axis_name="s")
  chunk = 128
  n_chunks = ROWS_PER_WORKER // chunk

  @functools.partial(
      pl.kernel,
      mesh=mesh,
      out_type=jax.ShapeDtypeStruct((BATCH,), jnp.float32),
      compiler_params=_COMPILER_PARAMS,
      scratch_types=[
          pltpu.VMEM((chunk, 2 * DIM), jnp.float32),
          pltpu.VMEM((chunk, 2 * DIM), jnp.float32),
          pltpu.VMEM((ROWS_PER_WORKER,), jnp.float32),
          pltpu.SemaphoreType.DMA,
      ],
  )
  def k(rows_u_hbm, rows_i_hbm, out_hbm, ubuf, ibuf, out_v, sem):
    wid = lax.axis_index("s") * NUM_CORES + lax.axis_index("c")
    base = wid * ROWS_PER_WORKER
    lanes = lax.iota(jnp.int32, LANES)

    def chunk_body(c, _):
      row0 = base + c * chunk
      cu = pltpu.async_copy(rows_u_hbm.at[pl.ds(row0, chunk)], ubuf, sem)
      ci = pltpu.async_copy(rows_i_hbm.at[pl.ds(row0, chunk)], ibuf, sem)
      cu.wait()
      ci.wait()

      def group_body(g, _g):
        j_vec = g * 16 + lanes
        acc = jnp.zeros((16,), jnp.float32)
        for d in range(DIM):
          col = (lanes + d) & (DIM - 1)
          ug = plsc.load_gather(ubuf, [j_vec, col])
          ig = plsc.load_gather(ibuf, [j_vec, col])
          acc = acc + ug * ig
        out_v[pl.ds(c * chunk + g * 16, 16)] = acc
        return _g
      lax.fori_loop(0, chunk // 16, group_body, 0, unroll=False)
      return _

    lax.fori_loop(0, n_chunks, chunk_body, 0, unroll=False)
    pltpu.sync_copy(out_v, out_hbm.at[pl.ds(base, ROWS_PER_WORKER)])

  return k


_extract_u = _make_extract(U_SIZE)
_extract_i = _make_extract(I_SIZE)
_dot = _make_dot()

_U_TS = (U_SIZE // 128) * 128
_I_TS = (I_SIZE // 128) * 128


@jax.jit
def kernel(users, items, user_emb, item_emb):
  tail_u = jnp.pad(user_emb[_U_TS:], ((0, 0), (0, DIM)))
  tail_i = jnp.pad(item_emb[_I_TS:], ((0, 0), (0, DIM)))
  rows_u = _extract_u(users, user_emb.T, tail_u)
  rows_i = _extract_i(items, item_emb.T, tail_i)
  return _dot(rows_u, rows_i)
